# baseline (speedup 1.0000x reference)
import functools

import jax
import jax.numpy as jnp
from jax import lax
from jax.experimental import pallas as pl
from jax.experimental.pallas import tpu as pltpu
from jax.experimental.pallas import tpu_sc as plsc

N = 100000
E = 1600000
K = 27
M = 16
NUM_CLASSES = 20
KM = K * M

NP = 100096
NR8 = NP // 8
NROW_REAL = N * M // 128

NTILES = 32
SCH = 128
GCH = 512
GPB = 7
EBLK = GPB * GCH
NB = 14
ECHUNK = NB * EBLK
EPAD = NTILES * ECHUNK
DUMMY_ROW = N + 8
ZROWS = NP // 16


def _sc_gather_scatter_body(z_hbm, gidx_hbm, dst_hbm, zeros_hbm, out_hbm,
                            gbuf, dbuf, rows, accum,
                            sr0, sr1, si0, si1, ss0, ss1, sz):
    cid = lax.axis_index("c")
    sid = lax.axis_index("s")
    wid = cid * 16 + sid
    sr = (sr0, sr1)
    si = (si0, si1)
    ss = (ss0, ss1)
    pltpu.async_copy(zeros_hbm.at[pl.ds(sid * ZROWS, ZROWS)],
                     accum.at[pl.ds(sid * ZROWS, ZROWS)], sz)

    NSC = GCH // SCH
    RPB = EBLK // SCH

    def fire_gather(islot, c, rslot):
        pltpu.async_copy(
            z_hbm.at[gbuf.at[islot, pl.ds(c * GCH, GCH)]],
            rows.at[rslot], sr[rslot])

    def wait_gather(islot, c, rslot):
        pltpu.make_async_copy(
            z_hbm.at[gbuf.at[islot, pl.ds(c * GCH, GCH)]],
            rows.at[rslot], sr[rslot]).wait()

    def fire_scatters(islot, c, rslot):
        for j in range(NSC):
            pltpu.async_copy(rows.at[rslot, pl.ds(j * SCH, SCH)],
                             accum.at[dbuf.at[islot, c * NSC + j]],
                             ss[rslot], add=True)

    def wait_scatters(islot, c, rslot):
        for j in range(NSC):
            pltpu.make_async_copy(rows.at[rslot, pl.ds(j * SCH, SCH)],
                                  accum.at[dbuf.at[islot, c * NSC + j]],
                                  ss[rslot]).wait()

    base = wid * NB * RPB
    ebase = wid * ECHUNK
    pltpu.sync_copy(gidx_hbm.at[pl.ds(ebase, EBLK)], gbuf.at[0])
    pltpu.sync_copy(dst_hbm.at[pl.ds(base, RPB)], dbuf.at[0])
    fire_gather(0, 0, 0)
    pltpu.make_async_copy(zeros_hbm.at[pl.ds(sid * ZROWS, ZROWS)],
                          accum.at[pl.ds(sid * ZROWS, ZROWS)], sz).wait()
    plsc.subcore_barrier()

    def pair_body(i, carry):
        for m in range(2 * GPB):
            s, c = (0, m) if m < GPB else (1, m - GPB)
            rs = m % 2
            rsn = 1 - rs
            bb = 2 * i + s

            if c == 0:
                @pl.when(bb + 1 < NB)
                def _():
                    pltpu.async_copy(
                        gidx_hbm.at[pl.ds(ebase + (bb + 1) * EBLK, EBLK)],
                        gbuf.at[1 - s], si[1 - s])
                    pltpu.async_copy(
                        dst_hbm.at[pl.ds(base + (bb + 1) * RPB, RPB)],
                        dbuf.at[1 - s], si[1 - s])

            if m == 0:
                @pl.when(i > 0)
                def _():
                    wait_scatters(1, GPB - 1, rsn)
                fire_gather(s, c + 1, rsn)
            elif c < GPB - 1:
                prev_s, prev_c = (s, c - 1) if c > 0 else (0, GPB - 1)
                wait_scatters(prev_s, prev_c, rsn)
                fire_gather(s, c + 1, rsn)
            else:
                wait_scatters(s, c - 1, rsn)

                @pl.when(bb + 1 < NB)
                def _():
                    pltpu.make_async_copy(
                        gidx_hbm.at[pl.ds(ebase + (bb + 1) * EBLK, EBLK)],
                        gbuf.at[1 - s], si[1 - s]).wait()
                    pltpu.make_async_copy(
                        dst_hbm.at[pl.ds(base + (bb + 1) * RPB, RPB)],
                        dbuf.at[1 - s], si[1 - s]).wait()
                    fire_gather(1 - s, 0, rsn)

            wait_gather(s, c, rs)
            fire_scatters(s, c, rs)
        return carry

    lax.fori_loop(0, NB // 2, pair_body, 0)
    wait_scatters(1, GPB - 1, 1)
    plsc.subcore_barrier()

    pltpu.sync_copy(accum.at[pl.ds(sid * ZROWS, ZROWS)],
                    out_hbm.at[cid, pl.ds(sid * ZROWS, ZROWS)])


_sc_gather_scatter = functools.partial(
    pl.kernel,
    out_type=jax.ShapeDtypeStruct((2, NP, M), jnp.float32),
    mesh=plsc.VectorSubcoreMesh(core_axis_name="c", subcore_axis_name="s"),
    scratch_types=[
        pltpu.VMEM((2, EBLK), jnp.int32),
        pltpu.VMEM((2, EBLK // SCH, SCH), jnp.int32),
        pltpu.VMEM((2, GCH, M), jnp.float32),
        pltpu.VMEM_SHARED((NP, M), jnp.float32),
        pltpu.SemaphoreType.DMA,
        pltpu.SemaphoreType.DMA,
        pltpu.SemaphoreType.DMA,
        pltpu.SemaphoreType.DMA,
        pltpu.SemaphoreType.DMA,
        pltpu.SemaphoreType.DMA,
        pltpu.SemaphoreType.DMA,
    ],
    compiler_params=pltpu.CompilerParams(use_tc_tiling_on_sc=False),
)(_sc_gather_scatter_body)


TB = 544


def _mm_body(x_ref, w_ref, o_ref):
    o = jnp.dot(x_ref[...], w_ref[...], preferred_element_type=jnp.float32)
    for t in range(K):
        o_ref[t] = o[:, t * 128:(t + 1) * 128]


def _stats_body(p_ref, o_ref):
    i = pl.program_id(0)

    @pl.when(i == 0)
    def _():
        o_ref[...] = jnp.zeros_like(o_ref)

    rid = lax.broadcasted_iota(jnp.int32, (TB, 128), 0) + i * TB
    msk = (rid < NROW_REAL).astype(jnp.float32)
    s = (p_ref[0] + p_ref[1]) * msk
    o_ref[...] += jnp.concatenate(
        [jnp.sum(s, 0, keepdims=True), jnp.sum(s * s, 0, keepdims=True)], 0)


def _fold16(v):
    acc = v[:, 0:16]
    for j in range(1, 8):
        acc = acc + v[:, 16 * j:16 * j + 16]
    return acc


def _tile128(v):
    return jnp.concatenate([v] * 8, axis=1)


def _bn_relu_128(p_ref, st_ref, g_ref, b_ref):
    s = p_ref[0] + p_ref[1]
    mu = _tile128(_fold16(st_ref[0:1, :]) * (1.0 / N))
    ex2 = _tile128(_fold16(st_ref[1:2, :]) * (1.0 / N))
    var = ex2 - mu * mu
    inv = lax.rsqrt(var + 1e-4)
    y = (s - mu) * inv * g_ref[...] + b_ref[...]
    return jnp.maximum(y, 0.0)


def _apply_body(p_ref, st_ref, g_ref, b_ref, w_ref, o_ref):
    y = _bn_relu_128(p_ref, st_ref, g_ref, b_ref)
    o = jnp.dot(y, w_ref[...], preferred_element_type=jnp.float32)
    for t in range(K):
        o_ref[t] = o[:, t * 128:(t + 1) * 128]


def _apply_fin_body(p_ref, st_ref, g_ref, b_ref, w_ref, bias_ref, o_ref):
    y = _bn_relu_128(p_ref, st_ref, g_ref, b_ref)
    o_ref[...] = (jnp.dot(y, w_ref[...], preferred_element_type=jnp.float32)
                  + bias_ref[...])


_mm0 = pl.pallas_call(
    _mm_body,
    grid=(NR8 // TB,),
    in_specs=[pl.BlockSpec((TB, 128), lambda i: (i, 0)),
              pl.BlockSpec((128, K * 128), lambda i: (0, 0))],
    out_specs=pl.BlockSpec((K, TB, 128), lambda i: (0, i, 0)),
    out_shape=jax.ShapeDtypeStruct((K, NR8, 128), jnp.float32),
)

_stats = pl.pallas_call(
    _stats_body,
    grid=(NR8 // TB,),
    in_specs=[pl.BlockSpec((2, TB, 128), lambda i: (0, i, 0))],
    out_specs=pl.BlockSpec((2, 128), lambda i: (0, 0)),
    out_shape=jax.ShapeDtypeStruct((2, 128), jnp.float32),
)

_apply_mid = pl.pallas_call(
    _apply_body,
    grid=(NR8 // TB,),
    in_specs=[pl.BlockSpec((2, TB, 128), lambda i: (0, i, 0)),
              pl.BlockSpec((2, 128), lambda i: (0, 0)),
              pl.BlockSpec((1, 128), lambda i: (0, 0)),
              pl.BlockSpec((1, 128), lambda i: (0, 0)),
              pl.BlockSpec((128, K * 128), lambda i: (0, 0))],
    out_specs=pl.BlockSpec((K, TB, 128), lambda i: (0, i, 0)),
    out_shape=jax.ShapeDtypeStruct((K, NR8, 128), jnp.float32),
)

_apply_fin = pl.pallas_call(
    _apply_fin_body,
    grid=(NR8 // TB,),
    in_specs=[pl.BlockSpec((2, TB, 128), lambda i: (0, i, 0)),
              pl.BlockSpec((2, 128), lambda i: (0, 0)),
              pl.BlockSpec((1, 128), lambda i: (0, 0)),
              pl.BlockSpec((1, 128), lambda i: (0, 0)),
              pl.BlockSpec((128, 8 * NUM_CLASSES), lambda i: (0, 0)),
              pl.BlockSpec((1, 8 * NUM_CLASSES), lambda i: (0, 0))],
    out_specs=pl.BlockSpec((TB, 8 * NUM_CLASSES), lambda i: (i, 0)),
    out_shape=jax.ShapeDtypeStruct((NR8, 8 * NUM_CLASSES), jnp.float32),
)


def _block_diag8(w):
    c = w.shape[1]
    return (jnp.eye(8, dtype=w.dtype)[:, None, :, None]
            * w[None, :, None, :]).reshape(128, 8 * c)


def kernel(feats, edge_index, kernel_id, W0, Ws, gammas, betas, Wlin, blin):
    src = edge_index[0]
    dst = edge_index[1]
    src2 = src.reshape(E // SCH, SCH)
    kid2 = kernel_id.reshape(E // SCH, SCH)
    q = (src2 & 7) * KM + kid2 * M
    gidx = ((q >> 7) * NP + (src2 >> 3) * 8 + ((q & 127) >> 4)).reshape(E)
    pad = EPAD - E
    gidx_p = jnp.concatenate([gidx, jnp.zeros((pad,), jnp.int32)])
    dst_p = jnp.concatenate([dst, jnp.full((pad,), DUMMY_ROW, jnp.int32)])
    dst2 = dst_p.reshape(EPAD // SCH, SCH)
    zeros_acc = jnp.zeros((NP, M), jnp.float32)

    feats16 = jnp.pad(feats, ((0, NP - N), (0, M - 3))).reshape(NR8, 128)
    w0f = jnp.pad(jnp.transpose(W0, (1, 0, 2)).reshape(3, KM),
                  ((0, M - 3), (0, 0)))
    wbd0 = _block_diag8(w0f)
    wbds = [_block_diag8(jnp.transpose(Ws[i], (1, 0, 2)).reshape(M, KM))
            for i in range(3)]
    wbd_fin = _block_diag8(Wlin)

    z = _mm0(feats16, wbd0)
    out = None
    for i in range(4):
        parts = _sc_gather_scatter(z.reshape(K * NP, M), gidx_p, dst2,
                                   zeros_acc)
        pview = parts.reshape(2, NR8, 128)
        st = _stats(pview)
        g = _tile128(gammas[i].reshape(1, M))
        b = _tile128(betas[i].reshape(1, M))
        if i < 3:
            z = _apply_mid(pview, st, g, b, wbds[i])
        else:
            bias8 = jnp.tile(blin.reshape(1, NUM_CLASSES), (1, 8))
            zf = _apply_fin(pview, st, g, b, wbd_fin, bias8)
            out = zf.reshape(NP, NUM_CLASSES)[:N]
    return out
